# baseline (speedup 1.0000x reference)
import functools

import jax
import jax.numpy as jnp
from jax import lax
from jax.experimental import pallas as pl
from jax.experimental.pallas import tpu as pltpu
from jax.experimental.pallas import tpu_sc as plsc

F = 26
V = 100000
H = 64
B = 16384
NC = 2
NS = 16
L = 16
NW = NC * NS
BPW = B // NW
CB = 128
NCHUNK = BPW // CB

_mesh = plsc.VectorSubcoreMesh(core_axis_name="c", subcore_axis_name="s")


@functools.partial(
    pl.kernel,
    out_type=jax.ShapeDtypeStruct((B, H), jnp.float32),
    mesh=_mesh,
    compiler_params=pltpu.CompilerParams(use_tc_tiling_on_sc=False),
    scratch_types=[
        pltpu.VMEM((F, BPW), jnp.int32),
        pltpu.VMEM((1, CB), jnp.int32),
        pltpu.VMEM((CB, H), jnp.float32),
        pltpu.VMEM((CB, H), jnp.float32),
        pltpu.SemaphoreType.DMA,
        pltpu.SemaphoreType.DMA,
    ],
)
def _cat_sum(xt_hbm, tab_hbm, out_hbm, xv, idxv, gbuf, acc, sem0, sem1):
    wid = lax.axis_index("s") * NC + lax.axis_index("c")
    base = wid * BPW
    pltpu.sync_copy(xt_hbm.at[:, pl.ds(base, BPW)], xv)

    for c in range(NCHUNK):
        cb = c * CB

        def make_idx(f, off):
            for s in range(CB // L):
                xc = xv[f, pl.ds(cb + s * L, L)]
                xc = jnp.minimum(jnp.maximum(xc, 0), V - 1)
                idxv[0, pl.ds(s * L, L)] = xc + off

        make_idx(0, jnp.int32(0))
        pltpu.async_copy(tab_hbm.at[idxv.at[0]], acc, sem0).wait()

        def fbody(f, carry):
            make_idx(f, f * V)
            pltpu.async_copy(tab_hbm.at[idxv.at[0]], gbuf, sem1).wait()

            def rbody(r, carry2):
                for cc in range(H // L):
                    g = gbuf[r, pl.ds(cc * L, L)]
                    plsc.addupdate(acc.at[r, pl.ds(cc * L, L)], g)
                return carry2

            lax.fori_loop(0, CB, rbody, 0, unroll=2)
            return carry

        lax.fori_loop(1, F, fbody, 0)

        pltpu.sync_copy(acc, out_hbm.at[pl.ds(base + cb, CB), :])


def kernel(x, tables):
    xt = jnp.transpose(x.astype(jnp.int32))
    tab = tables.reshape(F * V, H)
    return _cat_sum(xt, tab)
